# baseline (speedup 1.0000x reference)
import functools

import jax
import jax.numpy as jnp
from jax import lax
from jax.experimental import pallas as pl
from jax.experimental.pallas import tpu as pltpu
from jax.experimental.pallas import tpu_sc as plsc

B = 8
T_IN = 4096
D = 512
CTX = 5
T_OUT = 1365
CH = 39
CPB = T_OUT // CH
NCHUNK = B * CPB
SLAB = 3 * CH + 2
NW = 32
MAXC = -(-NCHUNK // NW)

_mesh = plsc.VectorSubcoreMesh(core_axis_name="c", subcore_axis_name="s")


@functools.partial(
    pl.kernel,
    mesh=_mesh,
    out_type=jax.ShapeDtypeStruct((B, T_OUT, CTX * D), jnp.float32),
    scratch_types=[
        pltpu.VMEM((SLAB, D), jnp.float32),
        pltpu.SemaphoreType.DMA,
        pltpu.SemaphoreType.DMA,
    ],
    compiler_params=pltpu.CompilerParams(use_tc_tiling_on_sc=False),
)
def _splice(feats_hbm, out_hbm, slab, lsem, wsem):
    nc = 2
    wid = lax.axis_index("s") * nc + lax.axis_index("c")

    def chunk_body(i, carry):
        c = wid + i * NW

        @pl.when(c < NCHUNK)
        def _():
            b = c // CPB
            j = c - b * CPB
            t0 = j * CH

            @pl.when(j == 0)
            def _():
                pltpu.async_copy(feats_hbm.at[b, pl.ds(0, SLAB - 2)],
                                 slab.at[pl.ds(2, SLAB - 2)], lsem).wait()
                pltpu.async_copy(feats_hbm.at[b, pl.ds(0, 1)],
                                 slab.at[pl.ds(0, 1)], lsem).wait()
                pltpu.async_copy(feats_hbm.at[b, pl.ds(0, 1)],
                                 slab.at[pl.ds(1, 1)], lsem).wait()

            @pl.when(j != 0)
            def _():
                pltpu.async_copy(feats_hbm.at[b, pl.ds(3 * t0 - 2, SLAB)],
                                 slab, lsem).wait()

            copies = []
            for tp in range(CH):
                for r in range(CTX):
                    copies.append(pltpu.async_copy(
                        slab.at[pl.ds(3 * tp + r, 1)],
                        out_hbm.at[b, pl.ds(t0 + tp, 1), pl.ds(r * D, D)],
                        wsem))
            for cp in copies:
                cp.wait()

        return carry

    lax.fori_loop(0, MAXC, chunk_body, 0)


def kernel(feats):
    return _splice(feats)


# speedup vs baseline: 2.0319x; 2.0319x over previous
import functools

import jax
import jax.numpy as jnp
from jax import lax
from jax.experimental import pallas as pl
from jax.experimental.pallas import tpu as pltpu
from jax.experimental.pallas import tpu_sc as plsc

B = 8
T_IN = 4096
D = 512
CTX = 5
T_OUT = 1365
CH = 16
CPB = T_OUT // CH
TAIL = T_OUT - CPB * CH
T0_TAIL = CPB * CH
NCHUNK = B * CPB
SLABR = 64
TSLABR = 24
R0_TAIL = T_IN - TSLABR
NW = 32
MAXC = -(-NCHUNK // NW)
LANES = 16

_mesh = plsc.VectorSubcoreMesh(core_axis_name="c", subcore_axis_name="s")


@functools.partial(
    pl.kernel,
    mesh=_mesh,
    out_type=jax.ShapeDtypeStruct((B, T_OUT, CTX * D), jnp.float32),
    scratch_types=[
        pltpu.VMEM((SLABR, D), jnp.float32),
        pltpu.VMEM((SLABR, D), jnp.float32),
        pltpu.VMEM((CH, CTX * D), jnp.float32),
        pltpu.VMEM((TAIL, CTX * D), jnp.float32),
        pltpu.SemaphoreType.DMA,
        pltpu.SemaphoreType.DMA,
        pltpu.SemaphoreType.DMA,
    ],
)
def _splice(feats_hbm, out_hbm, slab0, slab1, obuf, tbuf,
            lsem0, lsem1, wsem):
    nc = 2
    wid = lax.axis_index("s") * nc + lax.axis_index("c")

    def chunk_coords(k):
        c = wid + k * NW
        b = c // CPB
        j = c - b * CPB
        t0 = pl.multiple_of(j * CH, 8)
        r0a = pl.multiple_of(
            jnp.clip((3 * t0 - 2) // 8 * 8, 0, T_IN - SLABR), 8)
        return c, b, t0, r0a

    def start_load(k, slab, lsem):
        c, b, _, r0a = chunk_coords(k)

        @pl.when(c < NCHUNK)
        def _():
            pltpu.make_async_copy(feats_hbm.at[b, pl.ds(r0a, SLABR)],
                                  slab, lsem).start()

    def finish_chunk(k, slab, lsem):
        c, b, t0, r0a = chunk_coords(k)

        @pl.when(c < NCHUNK)
        def _():
            pltpu.make_async_copy(feats_hbm.at[b, pl.ds(r0a, SLABR)],
                                  slab, lsem).wait()

            def row_body(tp, carry):
                t = t0 + tp
                for rr in range(CTX):
                    row = jnp.maximum(3 * t + rr - 2, 0) - r0a
                    for cc in range(D // LANES):
                        obuf[tp, pl.ds(rr * D + cc * LANES, LANES)] = (
                            slab[row, pl.ds(cc * LANES, LANES)])
                return carry

            lax.fori_loop(0, CH, row_body, 0)
            pltpu.async_copy(obuf, out_hbm.at[b, pl.ds(t0, CH)],
                             wsem).wait()

    start_load(0, slab0, lsem0)
    start_load(1, slab1, lsem1)

    def body(i, carry):
        k0 = 2 * i
        finish_chunk(k0, slab0, lsem0)
        start_load(k0 + 2, slab0, lsem0)
        finish_chunk(k0 + 1, slab1, lsem1)
        start_load(k0 + 3, slab1, lsem1)
        return carry

    lax.fori_loop(0, MAXC // 2, body, 0)

    @pl.when(wid < B)
    def _():
        b = wid
        pltpu.async_copy(feats_hbm.at[b, pl.ds(R0_TAIL, TSLABR)],
                         slab0.at[pl.ds(0, TSLABR)], lsem0).wait()
        for tp in range(TAIL):
            t = T0_TAIL + tp
            for rr in range(CTX):
                row = 3 * t + rr - 2 - R0_TAIL
                for cc in range(D // LANES):
                    tbuf[tp, pl.ds(rr * D + cc * LANES, LANES)] = (
                        slab0[row, pl.ds(cc * LANES, LANES)])
        pltpu.async_copy(tbuf, out_hbm.at[b, pl.ds(T0_TAIL, TAIL)],
                         wsem).wait()


def kernel(feats):
    return _splice(feats)


# speedup vs baseline: 2.1566x; 1.0614x over previous
import functools

import jax
import jax.numpy as jnp
from jax import lax
from jax.experimental import pallas as pl
from jax.experimental.pallas import tpu as pltpu
from jax.experimental.pallas import tpu_sc as plsc

B = 8
T_IN = 4096
D = 512
CTX = 5
T_OUT = 1365
CH = 16
HF = CH // 2
CPB = T_OUT // CH
TAIL = T_OUT - CPB * CH
T0_TAIL = CPB * CH
NCHUNK = B * CPB
SLABR = 56
TSLABR = 24
R0_TAIL = T_IN - TSLABR
NW = 32
MAXC = -(-NCHUNK // NW)
LANES = 16

_mesh = plsc.VectorSubcoreMesh(core_axis_name="c", subcore_axis_name="s")


@functools.partial(
    pl.kernel,
    mesh=_mesh,
    out_type=jax.ShapeDtypeStruct((B, T_OUT, CTX * D), jnp.float32),
    scratch_types=[
        pltpu.VMEM((SLABR, D), jnp.float32),
        pltpu.VMEM((SLABR, D), jnp.float32),
        pltpu.VMEM((HF, CTX * D), jnp.float32),
        pltpu.VMEM((HF, CTX * D), jnp.float32),
        pltpu.VMEM((TAIL, CTX * D), jnp.float32),
        pltpu.SemaphoreType.DMA,
        pltpu.SemaphoreType.DMA,
        pltpu.SemaphoreType.DMA,
        pltpu.SemaphoreType.DMA,
    ],
)
def _splice(feats_hbm, out_hbm, slab0, slab1, obufa, obufb, tbuf,
            lsem0, lsem1, wsema, wsemb):
    nc = 2
    wid = lax.axis_index("s") * nc + lax.axis_index("c")

    def chunk_coords(k):
        c = wid + k * NW
        b = c // CPB
        j = c - b * CPB
        t0 = pl.multiple_of(j * CH, 8)
        r0a = pl.multiple_of(
            jnp.clip((3 * t0 - 2) // 8 * 8, 0, T_IN - SLABR), 8)
        return c, b, t0, r0a

    def start_load(k, slab, lsem):
        c, b, _, r0a = chunk_coords(k)

        @pl.when(c < NCHUNK)
        def _():
            pltpu.make_async_copy(feats_hbm.at[b, pl.ds(r0a, SLABR)],
                                  slab, lsem).start()

    def drain_write(k, obuf, wsem):
        c, _, _, _ = chunk_coords(k)

        @pl.when((k >= 0) & (c < NCHUNK))
        def _():
            pltpu.make_async_copy(out_hbm.at[0, pl.ds(0, HF)], obuf,
                                  wsem).wait()

    def assemble_half(slab, obuf, t0, th0, r0a):
        def row_body(tp, carry):
            t = t0 + th0 + tp
            for rr in range(CTX):
                row = jnp.maximum(3 * t + rr - 2, 0) - r0a
                for cc in range(D // LANES):
                    obuf[tp, pl.ds(rr * D + cc * LANES, LANES)] = (
                        slab[row, pl.ds(cc * LANES, LANES)])
            return carry

        lax.fori_loop(0, HF, row_body, 0)

    def process_chunk(k, slab, lsem):
        c, b, t0, r0a = chunk_coords(k)
        drain_write(k - 1, obufa, wsema)

        @pl.when(c < NCHUNK)
        def _():
            pltpu.make_async_copy(feats_hbm.at[b, pl.ds(r0a, SLABR)],
                                  slab, lsem).wait()
            assemble_half(slab, obufa, t0, 0, r0a)
            pltpu.make_async_copy(obufa, out_hbm.at[b, pl.ds(t0, HF)],
                                  wsema).start()

        drain_write(k - 1, obufb, wsemb)

        @pl.when(c < NCHUNK)
        def _():
            assemble_half(slab, obufb, t0, HF, r0a)
            pltpu.make_async_copy(obufb,
                                  out_hbm.at[b, pl.ds(t0 + HF, HF)],
                                  wsemb).start()

        start_load(k + 2, slab, lsem)

    start_load(0, slab0, lsem0)
    start_load(1, slab1, lsem1)

    def body(i, carry):
        process_chunk(2 * i, slab0, lsem0)
        process_chunk(2 * i + 1, slab1, lsem1)
        return carry

    lax.fori_loop(0, MAXC // 2, body, 0)
    drain_write(MAXC - 1, obufa, wsema)
    drain_write(MAXC - 1, obufb, wsemb)

    @pl.when(wid < B)
    def _():
        b = wid
        pltpu.async_copy(feats_hbm.at[b, pl.ds(R0_TAIL, TSLABR)],
                         slab0.at[pl.ds(0, TSLABR)], lsem0).wait()
        for tp in range(TAIL):
            t = T0_TAIL + tp
            for rr in range(CTX):
                row = 3 * t + rr - 2 - R0_TAIL
                for cc in range(D // LANES):
                    tbuf[tp, pl.ds(rr * D + cc * LANES, LANES)] = (
                        slab0[row, pl.ds(cc * LANES, LANES)])
        pltpu.async_copy(tbuf, out_hbm.at[b, pl.ds(T0_TAIL, TAIL)],
                         wsema).wait()


def kernel(feats):
    return _splice(feats)


# speedup vs baseline: 2.5794x; 1.1961x over previous
import functools

import jax
import jax.numpy as jnp
from jax import lax
from jax.experimental import pallas as pl
from jax.experimental.pallas import tpu as pltpu
from jax.experimental.pallas import tpu_sc as plsc

B = 8
T_IN = 4096
D = 512
CTX = 5
T_OUT = 1365
CH = 16
HF = CH // 2
CPB = T_OUT // CH
TAIL = T_OUT - CPB * CH
T0_TAIL = CPB * CH
NCHUNK = B * CPB
SLABR = 56
TSLABR = 24
R0_TAIL = T_IN - TSLABR
NW = 32
MAXC = -(-NCHUNK // NW)
LANES = 16

_mesh = plsc.VectorSubcoreMesh(core_axis_name="c", subcore_axis_name="s")


@functools.partial(
    pl.kernel,
    mesh=_mesh,
    out_type=jax.ShapeDtypeStruct((B, T_OUT, CTX * D), jnp.float32),
    scratch_types=[
        pltpu.VMEM((SLABR, D), jnp.float32),
        pltpu.VMEM((SLABR, D), jnp.float32),
        pltpu.VMEM((HF, CTX * D), jnp.float32),
        pltpu.VMEM((HF, CTX * D), jnp.float32),
        pltpu.VMEM((TAIL, CTX * D), jnp.float32),
        pltpu.SemaphoreType.DMA,
        pltpu.SemaphoreType.DMA,
        pltpu.SemaphoreType.DMA,
        pltpu.SemaphoreType.DMA,
    ],
)
def _splice(feats_hbm, out_hbm, slab0, slab1, obufa, obufb, tbuf,
            lsem0, lsem1, wsema, wsemb):
    nc = 2
    wid = lax.axis_index("s") * nc + lax.axis_index("c")

    def chunk_coords(k):
        c = wid + k * NW
        b = c // CPB
        j = c - b * CPB
        t0 = pl.multiple_of(j * CH, 8)
        r0a = pl.multiple_of(
            jnp.clip((3 * t0 - 2) // 8 * 8, 0, T_IN - SLABR), 8)
        return c, b, t0, r0a

    def start_load(k, slab, lsem):
        c, b, _, r0a = chunk_coords(k)

        @pl.when(c < NCHUNK)
        def _():
            pltpu.make_async_copy(feats_hbm.at[b, pl.ds(r0a, SLABR)],
                                  slab, lsem).start()

    def drain_write(k, obuf, wsem):
        c, _, _, _ = chunk_coords(k)

        @pl.when((k >= 0) & (c < NCHUNK))
        def _():
            pltpu.make_async_copy(out_hbm.at[0, pl.ds(0, HF)], obuf,
                                  wsem).wait()

    def assemble_half(slab, obuf, t0, th0, r0a):
        @plsc.parallel_loop(0, HF, unroll=2)
        def row_body(tp):
            t = t0 + th0 + tp
            for rr in range(CTX):
                row = jnp.maximum(3 * t + rr - 2, 0) - r0a
                for cc in range(D // LANES):
                    obuf[tp, pl.ds(rr * D + cc * LANES, LANES)] = (
                        slab[row, pl.ds(cc * LANES, LANES)])

    def process_chunk(k, slab, lsem):
        c, b, t0, r0a = chunk_coords(k)
        drain_write(k - 1, obufa, wsema)

        @pl.when(c < NCHUNK)
        def _():
            pltpu.make_async_copy(feats_hbm.at[b, pl.ds(r0a, SLABR)],
                                  slab, lsem).wait()
            assemble_half(slab, obufa, t0, 0, r0a)
            pltpu.make_async_copy(obufa, out_hbm.at[b, pl.ds(t0, HF)],
                                  wsema).start()

        drain_write(k - 1, obufb, wsemb)

        @pl.when(c < NCHUNK)
        def _():
            assemble_half(slab, obufb, t0, HF, r0a)
            pltpu.make_async_copy(obufb,
                                  out_hbm.at[b, pl.ds(t0 + HF, HF)],
                                  wsemb).start()

        start_load(k + 2, slab, lsem)

    start_load(0, slab0, lsem0)
    start_load(1, slab1, lsem1)

    def body(i, carry):
        process_chunk(2 * i, slab0, lsem0)
        process_chunk(2 * i + 1, slab1, lsem1)
        return carry

    lax.fori_loop(0, MAXC // 2, body, 0)
    drain_write(MAXC - 1, obufa, wsema)
    drain_write(MAXC - 1, obufb, wsemb)

    @pl.when(wid < B)
    def _():
        b = wid
        pltpu.async_copy(feats_hbm.at[b, pl.ds(R0_TAIL, TSLABR)],
                         slab0.at[pl.ds(0, TSLABR)], lsem0).wait()
        for tp in range(TAIL):
            t = T0_TAIL + tp
            for rr in range(CTX):
                row = 3 * t + rr - 2 - R0_TAIL
                for cc in range(D // LANES):
                    tbuf[tp, pl.ds(rr * D + cc * LANES, LANES)] = (
                        slab0[row, pl.ds(cc * LANES, LANES)])
        pltpu.async_copy(tbuf, out_hbm.at[b, pl.ds(T0_TAIL, TAIL)],
                         wsema).wait()


def kernel(feats):
    return _splice(feats)


# speedup vs baseline: 2.7948x; 1.0835x over previous
import functools

import jax
import jax.numpy as jnp
from jax import lax
from jax.experimental import pallas as pl
from jax.experimental.pallas import tpu as pltpu
from jax.experimental.pallas import tpu_sc as plsc

B = 8
T_IN = 4096
D = 512
CTX = 5
T_OUT = 1365
TR = 5
NCHUNK = T_OUT // TR
SR = 24
NW = 32
MAXC = -(-NCHUNK // NW)
LANES = 16

_mesh = plsc.VectorSubcoreMesh(core_axis_name="c", subcore_axis_name="s")


@functools.partial(
    pl.kernel,
    mesh=_mesh,
    out_type=jax.ShapeDtypeStruct((T_OUT, B, CTX * D), jnp.float32),
    scratch_types=[
        pltpu.VMEM((SR, D), jnp.float32),
        pltpu.VMEM((SR, D), jnp.float32),
        pltpu.VMEM((TR, B, CTX * D), jnp.float32),
        pltpu.SemaphoreType.DMA,
        pltpu.SemaphoreType.DMA,
        pltpu.SemaphoreType.DMA,
    ],
)
def _splice(feats_hbm, out_hbm, slab0, slab1, obuf, lsem0, lsem1, wsem):
    nc = 2
    wid = lax.axis_index("s") * nc + lax.axis_index("c")
    slabs = (slab0, slab1)
    lsems = (lsem0, lsem1)

    def chunk_coords(k):
        c = wid + k * NW
        t0 = c * TR
        r0a = pl.multiple_of(
            jnp.clip((3 * t0 - 2) // 8 * 8, 0, T_IN - SR), 8)
        return c, t0, r0a

    def start_load(k, b, slab, lsem):
        c, _, r0a = chunk_coords(k)

        @pl.when(c < NCHUNK)
        def _():
            pltpu.make_async_copy(feats_hbm.at[b, pl.ds(r0a, SR)],
                                  slab, lsem).start()

    def process_chunk(k):
        c, t0, r0a = chunk_coords(k)

        @pl.when((k >= 1) & (c - NW < NCHUNK))
        def _():
            pltpu.make_async_copy(out_hbm.at[pl.ds(0, TR)], obuf,
                                  wsem).wait()

        @pl.when(c < NCHUNK)
        def _():
            for b in range(B):
                pltpu.make_async_copy(feats_hbm.at[b, pl.ds(r0a, SR)],
                                      slabs[b % 2], lsems[b % 2]).wait()
                if b + 1 < B:
                    pltpu.make_async_copy(
                        feats_hbm.at[b + 1, pl.ds(r0a, SR)],
                        slabs[(b + 1) % 2], lsems[(b + 1) % 2]).start()
                slab = slabs[b % 2]

                def row_body(tp, carry):
                    t = t0 + tp
                    for rr in range(CTX):
                        row = jnp.maximum(3 * t + rr - 2, 0) - r0a
                        for cc in range(D // LANES):
                            obuf[tp, b,
                                 pl.ds(rr * D + cc * LANES, LANES)] = (
                                slab[row, pl.ds(cc * LANES, LANES)])
                    return carry

                lax.fori_loop(0, TR, row_body, 0)

            pltpu.make_async_copy(obuf, out_hbm.at[pl.ds(t0, TR)],
                                  wsem).start()

    def body(i, carry):
        k = i
        start_load(k, 0, slabs[0], lsems[0])
        process_chunk(k)
        return carry

    lax.fori_loop(0, MAXC, body, 0)

    c_last, _, _ = chunk_coords(MAXC - 1)

    @pl.when(c_last < NCHUNK)
    def _():
        pltpu.make_async_copy(out_hbm.at[pl.ds(0, TR)], obuf, wsem).wait()


def kernel(feats):
    o = _splice(feats)
    return jnp.transpose(o, (1, 0, 2))
